# Initial kernel scaffold; baseline (speedup 1.0000x reference)
import functools

import jax
import jax.numpy as jnp
from jax import lax
from jax.experimental import pallas as pl
from jax.experimental.pallas import tpu as pltpu
from jax.experimental.pallas import tpu_sc as plsc

N = 10000
DEG = 32
D1 = DEG + 1
F_IN = 128
F_OUT = 16
H = 8
HF = H * F_OUT
TW = HF + 2 * H

G = 8
NG = N // G
NC = 2
NS = 16
NW = NC * NS
QG, RG = divmod(NG, NW)


def _tc_project(x_ref, w_ref, b_ref, o_ref):
    o_ref[...] = (
        jnp.dot(x_ref[...], w_ref[...], preferred_element_type=jnp.float32)
        + b_ref[0:1, :]
    )


def _sc_gat(tab_hbm, adj_hbm, out_hbm, adj_v, self_v, rows_v, out_v, w_scr, sem):
    c = lax.axis_index("c")
    s = lax.axis_index("s")
    wid = s * NC + c
    ngrp = jnp.where(wid < RG, QG + 1, QG)
    base_g = wid * QG + jnp.minimum(wid, RG)
    lane = lax.broadcasted_iota(jnp.int32, (16,), 0)

    def grp_body(gi, carry):
        b = (base_g + gi) * G
        pltpu.sync_copy(adj_hbm.at[pl.ds(b, G)], adj_v)
        pltpu.sync_copy(tab_hbm.at[pl.ds(b, G)], self_v)
        descs = [
            pltpu.async_copy(
                tab_hbm.at[adj_v.at[i]], rows_v.at[pl.ds(i * DEG, DEG)], sem
            )
            for i in range(G)
        ]
        for d_ in descs:
            d_.wait()

        def node_body(i, carry2):
            for h in range(H):
                cself = self_v[i, HF + H + h]
                row0 = i * DEG
                ridx1 = row0 + lane
                ridx2 = ridx1 + 16
                colv = jnp.full((16,), HF + h, dtype=jnp.int32)
                sn1 = plsc.load_gather(rows_v, [ridx1, colv])
                sn2 = plsc.load_gather(rows_v, [ridx2, colv])
                a1 = sn1 + cself
                a2 = sn2 + cself
                a1 = jnp.where(a1 > 0, a1, 0.2 * a1)
                a2 = jnp.where(a2 > 0, a2, 0.2 * a2)
                asf = self_v[i, HF + h] + cself
                asf = jnp.where(asf > 0, asf, 0.2 * asf)
                m = jnp.maximum(jnp.max(a1), jnp.max(a2))
                m = jnp.maximum(m, asf)
                e1 = jnp.exp(a1 - m)
                e2 = jnp.exp(a2 - m)
                esv = jnp.exp(jnp.where(lane == 0, asf - m, jnp.float32(-1e30)))
                es = jnp.sum(esv)
                z = jnp.sum(e1) + jnp.sum(e2) + es
                w_scr[pl.ds(0, 16)] = e1
                w_scr[pl.ds(16, 16)] = e2
                acc0 = es * self_v[i, pl.ds(h * F_OUT, F_OUT)]
                acc1 = jnp.zeros((16,), jnp.float32)
                acc2 = jnp.zeros((16,), jnp.float32)
                acc3 = jnp.zeros((16,), jnp.float32)
                accs = [acc0, acc1, acc2, acc3]
                for d in range(DEG):
                    wgt = w_scr[d]
                    accs[d % 4] = accs[d % 4] + wgt * rows_v[
                        row0 + d, pl.ds(h * F_OUT, F_OUT)
                    ]
                acc = (accs[0] + accs[1]) + (accs[2] + accs[3])
                out_v[i, pl.ds(h * F_OUT, F_OUT)] = acc / z
            return carry2

        lax.fori_loop(0, G, node_body, 0)
        pltpu.sync_copy(out_v, out_hbm.at[pl.ds(b, G)])
        return carry

    lax.fori_loop(0, ngrp, grp_body, 0)


def kernel(adjlist, features, W_w, W_b, a_w, a_b):
    adjlist = adjlist.astype(jnp.int32)
    a_w_self = a_w[:, :F_OUT]
    a_w_neigh = a_w[:, F_OUT:]
    eye = jnp.eye(H, dtype=jnp.float32)
    An = (a_w_neigh[:, :, None] * eye[:, None, :]).reshape(HF, H)
    As = (a_w_self[:, :, None] * eye[:, None, :]).reshape(HF, H)
    Wt = W_w.T
    Wcat = jnp.concatenate([Wt, Wt @ An, Wt @ As], axis=1)
    bcat = jnp.concatenate([W_b, W_b @ An, W_b @ As + a_b])
    bcat8 = jnp.broadcast_to(bcat[None, :], (8, TW))

    rows_blk = 1000
    table = pl.pallas_call(
        _tc_project,
        grid=(N // rows_blk,),
        in_specs=[
            pl.BlockSpec((rows_blk, F_IN), lambda i: (i, 0)),
            pl.BlockSpec((F_IN, TW), lambda i: (0, 0)),
            pl.BlockSpec((8, TW), lambda i: (0, 0)),
        ],
        out_specs=pl.BlockSpec((rows_blk, TW), lambda i: (i, 0)),
        out_shape=jax.ShapeDtypeStruct((N, TW), jnp.float32),
    )(features, Wcat, bcat8)

    mesh = plsc.VectorSubcoreMesh(core_axis_name="c", subcore_axis_name="s")
    sc = functools.partial(
        pl.kernel,
        out_type=jax.ShapeDtypeStruct((N, HF), jnp.float32),
        mesh=mesh,
        scratch_types=[
            pltpu.VMEM((G, DEG), jnp.int32),
            pltpu.VMEM((G, TW), jnp.float32),
            pltpu.VMEM((G * DEG, TW), jnp.float32),
            pltpu.VMEM((G, HF), jnp.float32),
            pltpu.VMEM((DEG,), jnp.float32),
            pltpu.SemaphoreType.DMA,
        ],
    )(_sc_gat)
    return sc(table, adjlist)


# speedup vs baseline: 3.6126x; 3.6126x over previous
import functools

import jax
import jax.numpy as jnp
from jax import lax
from jax.experimental import pallas as pl
from jax.experimental.pallas import tpu as pltpu
from jax.experimental.pallas import tpu_sc as plsc

N = 10000
DEG = 32
D1 = DEG + 1
F_IN = 128
F_OUT = 16
H = 8
HF = H * F_OUT
TW = HF + 2 * H

G = 8
NG = N // G
NC = 2
NS = 16
NW = NC * NS
QG, RG = divmod(NG, NW)


def _tc_project(x_ref, w_ref, b_ref, o_ref):
    o_ref[...] = (
        jnp.dot(x_ref[...], w_ref[...], preferred_element_type=jnp.float32)
        + b_ref[0:1, :]
    )


def _sc_gat(tab_hbm, adj_hbm, out_hbm, adj_v, self_v, rows_v, out_v, sem):
    c = lax.axis_index("c")
    s = lax.axis_index("s")
    wid = s * NC + c
    ngrp = jnp.where(wid < RG, QG + 1, QG)
    base_g = wid * QG + jnp.minimum(wid, RG)
    lane = lax.broadcasted_iota(jnp.int32, (16,), 0)

    def grp_body(gi, carry):
        b = (base_g + gi) * G
        pltpu.sync_copy(adj_hbm.at[pl.ds(b, G)], adj_v)
        pltpu.sync_copy(tab_hbm.at[pl.ds(b, G)], self_v)
        descs = [
            pltpu.async_copy(
                tab_hbm.at[adj_v.at[i]], rows_v.at[pl.ds(i * DEG, DEG)], sem
            )
            for i in range(G)
        ]
        for d_ in descs:
            d_.wait()

        def node_body(i, carry2):
            srow = self_v[i, pl.ds(HF, 2 * H)]
            row0 = i * DEG
            for h in range(H):
                cself = srow[H + h]
                ridx1 = row0 + lane
                ridx2 = ridx1 + 16
                colv = jnp.full((16,), HF + h, dtype=jnp.int32)
                sn1 = plsc.load_gather(rows_v, [ridx1, colv])
                sn2 = plsc.load_gather(rows_v, [ridx2, colv])
                a1 = sn1 + cself
                a2 = sn2 + cself
                a1 = jnp.where(a1 > 0, a1, 0.2 * a1)
                a2 = jnp.where(a2 > 0, a2, 0.2 * a2)
                asf = srow[h] + cself
                asf = jnp.where(asf > 0, asf, 0.2 * asf)
                m = jnp.maximum(jnp.max(a1), jnp.max(a2))
                m = jnp.maximum(m, asf)
                e1 = jnp.exp(a1 - m)
                e2 = jnp.exp(a2 - m)
                esv = jnp.exp(jnp.where(lane == 0, asf - m, jnp.float32(-1e30)))
                es = jnp.sum(esv)
                z = jnp.sum(e1) + jnp.sum(e2) + es
                acc0 = es * self_v[i, pl.ds(h * F_OUT, F_OUT)]
                acc1 = jnp.zeros((16,), jnp.float32)
                acc2 = jnp.zeros((16,), jnp.float32)
                acc3 = jnp.zeros((16,), jnp.float32)
                accs = [acc0, acc1, acc2, acc3]
                for d in range(DEG):
                    wgt = e1[d] if d < 16 else e2[d - 16]
                    accs[d % 4] = accs[d % 4] + wgt * rows_v[
                        row0 + d, pl.ds(h * F_OUT, F_OUT)
                    ]
                acc = (accs[0] + accs[1]) + (accs[2] + accs[3])
                out_v[i, pl.ds(h * F_OUT, F_OUT)] = acc / z
            return carry2

        lax.fori_loop(0, G, node_body, 0)
        pltpu.sync_copy(out_v, out_hbm.at[pl.ds(b, G)])
        return carry

    lax.fori_loop(0, ngrp, grp_body, 0)


def kernel(adjlist, features, W_w, W_b, a_w, a_b):
    adjlist = adjlist.astype(jnp.int32)
    a_w_self = a_w[:, :F_OUT]
    a_w_neigh = a_w[:, F_OUT:]
    eye = jnp.eye(H, dtype=jnp.float32)
    An = (a_w_neigh[:, :, None] * eye[:, None, :]).reshape(HF, H)
    As = (a_w_self[:, :, None] * eye[:, None, :]).reshape(HF, H)
    Wt = W_w.T
    Wcat = jnp.concatenate([Wt, Wt @ An, Wt @ As], axis=1)
    bcat = jnp.concatenate([W_b, W_b @ An, W_b @ As + a_b])
    bcat8 = jnp.broadcast_to(bcat[None, :], (8, TW))

    rows_blk = 1000
    table = pl.pallas_call(
        _tc_project,
        grid=(N // rows_blk,),
        in_specs=[
            pl.BlockSpec((rows_blk, F_IN), lambda i: (i, 0)),
            pl.BlockSpec((F_IN, TW), lambda i: (0, 0)),
            pl.BlockSpec((8, TW), lambda i: (0, 0)),
        ],
        out_specs=pl.BlockSpec((rows_blk, TW), lambda i: (i, 0)),
        out_shape=jax.ShapeDtypeStruct((N, TW), jnp.float32),
    )(features, Wcat, bcat8)

    mesh = plsc.VectorSubcoreMesh(core_axis_name="c", subcore_axis_name="s")
    sc = functools.partial(
        pl.kernel,
        out_type=jax.ShapeDtypeStruct((N, HF), jnp.float32),
        mesh=mesh,
        scratch_types=[
            pltpu.VMEM((G, DEG), jnp.int32),
            pltpu.VMEM((G, TW), jnp.float32),
            pltpu.VMEM((G * DEG, TW), jnp.float32),
            pltpu.VMEM((G, HF), jnp.float32),
            pltpu.SemaphoreType.DMA,
        ],
        compiler_params=pltpu.CompilerParams(
            use_tc_tiling_on_sc=False, needs_layout_passes=False
        ),
    )(_sc_gat)
    return sc(table, adjlist)
